# Initial kernel scaffold; baseline (speedup 1.0000x reference)
import jax
import jax.numpy as jnp
from jax.experimental import pallas as pl


def kernel(x, shift_indices):
    raise NotImplementedError("write your pallas kernel here")


# speedup vs baseline: 2.4823x; 2.4823x over previous
import jax
import jax.numpy as jnp
from jax.experimental import pallas as pl

_CBLK = 8


def _shift_kernel(idx_ref, x_ref, o_ref):
    xv = x_ref[0]
    idx = idx_ref[...].astype(jnp.int32)
    idxb = jnp.broadcast_to(idx[:, None, :], xv.shape)
    o_ref[0] = jnp.take_along_axis(xv, idxb, axis=-1)


def kernel(x, shift_indices):
    B, C, T, V = x.shape
    cblk = _CBLK
    grid = (B, C // cblk)
    return pl.pallas_call(
        _shift_kernel,
        grid=grid,
        in_specs=[
            pl.BlockSpec((cblk, V), lambda b, j: (j, 0)),
            pl.BlockSpec((1, cblk, T, V), lambda b, j: (b, j, 0, 0)),
        ],
        out_specs=pl.BlockSpec((1, cblk, T, V), lambda b, j: (b, j, 0, 0)),
        out_shape=jax.ShapeDtypeStruct((B, C, T, V), x.dtype),
    )(shift_indices.astype(jnp.int32), x)
